# baseline (speedup 1.0000x reference)
import functools

import jax
import jax.numpy as jnp
from jax import lax
from jax.experimental import pallas as pl
from jax.experimental.pallas import tpu as pltpu
from jax.experimental.pallas import tpu_sc as plsc

NC = 2
NS = 16
L = 16
NW = NC * NS

_HIGH = jax.lax.Precision.HIGHEST


def _mesh():
    return plsc.VectorSubcoreMesh(
        core_axis_name="c", subcore_axis_name="s", num_cores=NC, num_subcores=NS
    )


def _fill1d(ref, n, value):
    v = jnp.full((L,), value, jnp.float32)

    def body(i, carry):
        ref[pl.ds(i * L, L)] = v
        return carry

    lax.fori_loop(0, n // L, body, 0)


def _fill2d(ref, rows, cols, value):
    v = jnp.full((L,), value, jnp.float32)
    per_row = cols // L

    def body(i, carry):
        r = i // per_row
        j = i % per_row
        ref[r, pl.ds(j * L, L)] = v
        return carry

    lax.fori_loop(0, rows * per_row, body, 0)


def _make_deg_kernel(e_total, n_nodes):
    ept = e_total // NW
    chunk = 80
    assert ept % chunk == 0
    zlen = 2000
    assert n_nodes % zlen == 0

    nb = 5
    nchunks = ept // chunk
    assert nchunks % nb == 0

    @functools.partial(
        pl.kernel,
        out_type=jax.ShapeDtypeStruct((NC, n_nodes), jnp.float32),
        mesh=_mesh(),
        scratch_types=[
            [pltpu.VMEM((chunk,), jnp.int32) for _ in range(nb)],
            pltpu.VMEM((chunk,), jnp.float32),
            pltpu.VMEM((zlen,), jnp.float32),
            pltpu.VMEM_SHARED((n_nodes,), jnp.float32),
            [pltpu.SemaphoreType.DMA for _ in range(nb)],
            [pltpu.SemaphoreType.DMA for _ in range(nb)],
        ],
    )
    def deg_kernel(dst_hbm, out_hbm, didx, ones, zbuf, acc, isem, ssem):
        c = lax.axis_index("c")
        s = lax.axis_index("s")
        wid = c * NS + s

        _fill1d(ones, chunk, 1.0)

        @pl.when(s == 0)
        def _():
            _fill1d(zbuf, zlen, 0.0)

            def zcp(k, carry):
                pltpu.sync_copy(zbuf, acc.at[pl.ds(k * zlen, zlen)])
                return carry

            lax.fori_loop(0, n_nodes // zlen, zcp, 0)

        plsc.subcore_barrier()

        base = wid * ept

        def body(t, carry):
            offs = [base + (t * nb + b) * chunk for b in range(nb)]
            idesc = [
                pltpu.async_copy(
                    dst_hbm.at[pl.ds(offs[b], chunk)], didx[b], isem[b])
                for b in range(nb)
            ]
            sdesc = []
            for b in range(nb):
                idesc[b].wait()
                sdesc.append(
                    pltpu.async_copy(ones, acc.at[didx[b]], ssem[b],
                                     add=True))
            for b in range(nb):
                sdesc[b].wait()
            return carry

        lax.fori_loop(0, nchunks // nb, body, 0)

        plsc.subcore_barrier()

        @pl.when(s == 0)
        def _():
            pltpu.sync_copy(acc, out_hbm.at[c])

    return deg_kernel


def _make_agg_kernel(e_total, n_nodes, d, tc_tiling=True):
    ept = e_total // NW
    chunk = 80
    assert ept % chunk == 0
    nb = 4
    nchunks = ept // chunk
    ngroups = nchunks // nb
    n_rchunks = n_nodes // chunk
    rounds = (n_rchunks + NS - 1) // NS

    @functools.partial(
        pl.kernel,
        out_type=jax.ShapeDtypeStruct((NC, n_nodes, d), jnp.float32),
        mesh=_mesh(),
        compiler_params=pltpu.CompilerParams(use_tc_tiling_on_sc=tc_tiling),
        scratch_types=[
            [pltpu.VMEM((chunk,), jnp.int32) for _ in range(nb)],
            [pltpu.VMEM((chunk,), jnp.int32) for _ in range(nb)],
            [pltpu.VMEM((chunk, d), jnp.float32) for _ in range(nb)],
            pltpu.VMEM_SHARED((n_nodes, d), jnp.float32),
            [pltpu.SemaphoreType.DMA for _ in range(nb)],
            [pltpu.SemaphoreType.DMA for _ in range(nb)],
            [pltpu.SemaphoreType.DMA for _ in range(nb)],
        ],
    )
    def agg_kernel(h_hbm, src_hbm, dst_hbm, out_hbm, sidx, didx, rows,
                   acc, isem, gsem, ssem):
        c = lax.axis_index("c")
        s = lax.axis_index("s")
        wid = c * NS + s

        _fill2d(rows[0], chunk, d, 0.0)

        def zcp(k, carry):
            cid = s + k * NS

            @pl.when(cid < n_rchunks)
            def _():
                pltpu.sync_copy(rows[0], acc.at[pl.ds(cid * chunk, chunk)])

            return carry

        lax.fori_loop(0, rounds, zcp, 0)
        plsc.subcore_barrier()

        base = wid * ept

        def do_group(offs):
            k = len(offs)
            idesc = []
            for b in range(k):
                idesc.append((
                    pltpu.async_copy(
                        src_hbm.at[pl.ds(offs[b], chunk)], sidx[b], isem[b]),
                    pltpu.async_copy(
                        dst_hbm.at[pl.ds(offs[b], chunk)], didx[b], isem[b]),
                ))
            gdesc = []
            for b in range(k):
                idesc[b][0].wait()
                idesc[b][1].wait()
                gdesc.append(
                    pltpu.async_copy(h_hbm.at[sidx[b]], rows[b], gsem[b]))
            sdesc = []
            for b in range(k):
                gdesc[b].wait()
                sdesc.append(
                    pltpu.async_copy(rows[b], acc.at[didx[b]], ssem[b],
                                     add=True))
            for b in range(k):
                sdesc[b].wait()

        def body(t, carry):
            do_group([base + (t * nb + b) * chunk for b in range(nb)])
            return carry

        lax.fori_loop(0, ngroups, body, 0)
        if nchunks % nb:
            do_group([base + g * chunk
                      for g in range(ngroups * nb, nchunks)])

        plsc.subcore_barrier()

        def ocp(k, carry):
            cid = s + k * NS

            @pl.when(cid < n_rchunks)
            def _():
                r0 = cid * chunk
                pltpu.sync_copy(
                    acc.at[pl.ds(r0, chunk)], out_hbm.at[c, pl.ds(r0, chunk)]
                )

            return carry

        lax.fori_loop(0, rounds, ocp, 0)

    return agg_kernel


_R = 1000


def _row_specs(n, shapes):
    specs = []
    for shp in shapes:
        if shp[0] == n:
            specs.append(pl.BlockSpec((_R, shp[1]), lambda i: (i, 0)))
        else:
            specs.append(pl.BlockSpec(shp, lambda i: (0, 0)))
    return specs


def _tc_prep(x, w1, deg0, deg1):
    n = x.shape[0]
    d_out = w1.shape[1]

    def body(x_ref, w_ref, d0_ref, d1_ref, dinv_ref, hs_ref):
        dinv = lax.rsqrt(d0_ref[...] + d1_ref[...] + 1.0)
        dinv_ref[...] = dinv
        h = jnp.dot(x_ref[...], w_ref[...], precision=_HIGH,
                    preferred_element_type=jnp.float32)
        hs_ref[...] = h * dinv

    return pl.pallas_call(
        body,
        grid=(n // _R,),
        in_specs=_row_specs(n, [x.shape, w1.shape, deg0.shape, deg1.shape]),
        out_specs=(
            pl.BlockSpec((_R, 1), lambda i: (i, 0)),
            pl.BlockSpec((_R, d_out), lambda i: (i, 0)),
        ),
        out_shape=(
            jax.ShapeDtypeStruct((n, 1), jnp.float32),
            jax.ShapeDtypeStruct((n, d_out), jnp.float32),
        ),
    )(x, w1, deg0, deg1)


def _tc_mid(agg0, agg1, hs, dinv, b1, w2):
    n, d = hs.shape

    def body(a0_ref, a1_ref, hs_ref, dinv_ref, b1_ref, w2_ref, o_ref):
        z = dinv_ref[...] * (a0_ref[...] + a1_ref[...] + hs_ref[...])
        z = jnp.maximum(z + b1_ref[...], 0.0)
        o_ref[...] = dinv_ref[...] * jnp.dot(
            z, w2_ref[...], precision=_HIGH, preferred_element_type=jnp.float32
        )

    return pl.pallas_call(
        body,
        grid=(n // _R,),
        in_specs=_row_specs(
            n, [agg0.shape, agg1.shape, hs.shape, dinv.shape, b1.shape,
                w2.shape]),
        out_specs=pl.BlockSpec((_R, w2.shape[1]), lambda i: (i, 0)),
        out_shape=jax.ShapeDtypeStruct((n, w2.shape[1]), jnp.float32),
    )(agg0, agg1, hs, dinv, b1, w2)


def _tc_final(agg0, agg1, h2s, dinv, b2):
    n = h2s.shape[0]
    n_cls = b2.shape[1]

    def body(a0_ref, a1_ref, hs_ref, dinv_ref, b2_ref, o_ref):
        o = dinv_ref[...] * (
            a0_ref[:, :n_cls] + a1_ref[:, :n_cls] + hs_ref[:, :n_cls]
        )
        o = o + b2_ref[...]
        m = jnp.max(o, axis=1, keepdims=True)
        sh = o - m
        lse = jnp.log(jnp.sum(jnp.exp(sh), axis=1, keepdims=True))
        o_ref[...] = sh - lse

    return pl.pallas_call(
        body,
        grid=(n // _R,),
        in_specs=_row_specs(
            n, [agg0.shape, agg1.shape, h2s.shape, dinv.shape, b2.shape]),
        out_specs=pl.BlockSpec((_R, n_cls), lambda i: (i, 0)),
        out_shape=jax.ShapeDtypeStruct((n, n_cls), jnp.float32),
    )(agg0, agg1, h2s, dinv, b2)


def kernel(x, edge_index, W1, b1, W2, b2):
    n, d_in = x.shape
    e_total = edge_index.shape[1]
    d_hid = W1.shape[1]
    n_cls = W2.shape[1]

    src = edge_index[0].astype(jnp.int32)
    dst = edge_index[1].astype(jnp.int32)

    deg_k = _make_deg_kernel(e_total, n)
    agg1_k = _make_agg_kernel(e_total, n, d_hid)
    agg2_k = _make_agg_kernel(e_total, n, n_cls, tc_tiling=False)

    degp = deg_k(dst)
    dinv, h1s = _tc_prep(x, W1, degp[0].reshape(n, 1), degp[1].reshape(n, 1))
    aggp1 = agg1_k(h1s, src, dst)
    h2s = _tc_mid(aggp1[0], aggp1[1], h1s, dinv, b1.reshape(1, d_hid), W2)
    aggp2 = agg2_k(h2s, src, dst)
    return _tc_final(aggp2[0], aggp2[1], h2s, dinv, b2.reshape(1, n_cls))


# speedup vs baseline: 1.4015x; 1.4015x over previous
import functools

import jax
import jax.numpy as jnp
from jax import lax
from jax.experimental import pallas as pl
from jax.experimental.pallas import tpu as pltpu
from jax.experimental.pallas import tpu_sc as plsc

NC = 2
NS = 16
L = 16
NW = NC * NS

_HIGH = jax.lax.Precision.HIGHEST


def _mesh():
    return plsc.VectorSubcoreMesh(
        core_axis_name="c", subcore_axis_name="s", num_cores=NC, num_subcores=NS
    )


def _fill1d(ref, n, value):
    v = jnp.full((L,), value, jnp.float32)

    def body(i, carry):
        ref[pl.ds(i * L, L)] = v
        return carry

    lax.fori_loop(0, n // L, body, 0)


def _fill2d(ref, rows, cols, value):
    v = jnp.full((L,), value, jnp.float32)
    per_row = cols // L

    def body(i, carry):
        r = i // per_row
        j = i % per_row
        ref[r, pl.ds(j * L, L)] = v
        return carry

    lax.fori_loop(0, rows * per_row, body, 0)


def _make_deg_kernel(e_total, n_nodes):
    ept = e_total // NW
    chunk = 80
    assert ept % chunk == 0
    nchunks = ept // chunk
    zlen = 2000
    assert n_nodes % zlen == 0
    K = 8

    @functools.partial(
        pl.kernel,
        out_type=jax.ShapeDtypeStruct((NC, n_nodes), jnp.float32),
        mesh=_mesh(),
        compiler_params=pltpu.CompilerParams(use_tc_tiling_on_sc=False),
        scratch_types=[
            pltpu.VMEM((nchunks, chunk), jnp.int32),
            pltpu.VMEM((chunk,), jnp.float32),
            pltpu.VMEM((zlen,), jnp.float32),
            pltpu.VMEM_SHARED((n_nodes,), jnp.float32),
            pltpu.SemaphoreType.DMA,
        ],
    )
    def deg_kernel(dst3_hbm, out_hbm, di_all, ones, zbuf, acc, ssem):
        c = lax.axis_index("c")
        s = lax.axis_index("s")
        wid = c * NS + s

        _fill1d(ones, chunk, 1.0)

        @pl.when(s == 0)
        def _():
            _fill1d(zbuf, zlen, 0.0)

            def zcp(k, carry):
                pltpu.sync_copy(zbuf, acc.at[pl.ds(k * zlen, zlen)])
                return carry

            lax.fori_loop(0, n_nodes // zlen, zcp, 0)

        pltpu.sync_copy(dst3_hbm.at[wid], di_all)
        plsc.subcore_barrier()

        def body(g, carry):
            pltpu.async_copy(ones, acc.at[di_all.at[g]], ssem, add=True)

            @pl.when(g >= K)
            def _():
                pltpu.make_async_copy(ones, acc.at[di_all.at[0]], ssem).wait()

            return carry

        lax.fori_loop(0, nchunks, body, 0)
        for _ in range(min(K, nchunks)):
            pltpu.make_async_copy(ones, acc.at[di_all.at[0]], ssem).wait()

        plsc.subcore_barrier()

        @pl.when(s == 0)
        def _():
            pltpu.sync_copy(acc, out_hbm.at[c])

    return deg_kernel


def _make_agg_kernel(e_total, n_nodes, d):
    ept = e_total // NW
    chunk = 80
    assert ept % chunk == 0
    nchunks = ept // chunk
    nb = 3 if d > 64 else 5
    ngroups = nchunks // nb
    n_rchunks = n_nodes // chunk
    rounds = (n_rchunks + NS - 1) // NS

    @functools.partial(
        pl.kernel,
        out_type=jax.ShapeDtypeStruct((NC, n_nodes, d), jnp.float32),
        mesh=_mesh(),
        compiler_params=pltpu.CompilerParams(use_tc_tiling_on_sc=False),
        scratch_types=[
            pltpu.VMEM((nchunks, chunk), jnp.int32),
            pltpu.VMEM((nchunks, chunk), jnp.int32),
            [pltpu.VMEM((chunk, d), jnp.float32) for _ in range(nb)],
            pltpu.VMEM_SHARED((n_nodes, d), jnp.float32),
            [pltpu.SemaphoreType.DMA for _ in range(nb)],
            [pltpu.SemaphoreType.DMA for _ in range(nb)],
        ],
    )
    def agg_kernel(h_hbm, src3_hbm, dst3_hbm, out_hbm, si_all, di_all, rows,
                   acc, gsem, ssem):
        c = lax.axis_index("c")
        s = lax.axis_index("s")
        wid = c * NS + s

        _fill2d(rows[0], chunk, d, 0.0)

        def zcp(k, carry):
            cid = s + k * NS

            @pl.when(cid < n_rchunks)
            def _():
                pltpu.sync_copy(rows[0], acc.at[pl.ds(cid * chunk, chunk)])

            return carry

        lax.fori_loop(0, rounds, zcp, 0)
        pltpu.sync_copy(src3_hbm.at[wid], si_all)
        pltpu.sync_copy(dst3_hbm.at[wid], di_all)
        plsc.subcore_barrier()

        for b in range(nb):
            pltpu.async_copy(h_hbm.at[si_all.at[b]], rows[b], gsem[b])

        def visit(g, b):
            pltpu.make_async_copy(
                h_hbm.at[si_all.at[0]], rows[b], gsem[b]).wait()
            pltpu.async_copy(rows[b], acc.at[di_all.at[g]], ssem[b],
                             add=True)

            @pl.when(g + nb < nchunks)
            def _():
                pltpu.make_async_copy(
                    rows[b], acc.at[di_all.at[0]], ssem[b]).wait()
                pltpu.async_copy(
                    h_hbm.at[si_all.at[g + nb]], rows[b], gsem[b])

        def body(t, carry):
            for b in range(nb):
                visit(t * nb + b, b)
            return carry

        lax.fori_loop(0, ngroups, body, 0)
        for i in range(nchunks % nb):
            visit(ngroups * nb + i, i)

        for b in range(nb):
            pltpu.make_async_copy(rows[b], acc.at[di_all.at[0]], ssem[b]).wait()

        plsc.subcore_barrier()

        def ocp(k, carry):
            cid = s + k * NS

            @pl.when(cid < n_rchunks)
            def _():
                r0 = cid * chunk
                pltpu.sync_copy(
                    acc.at[pl.ds(r0, chunk)], out_hbm.at[c, pl.ds(r0, chunk)]
                )

            return carry

        lax.fori_loop(0, rounds, ocp, 0)

    return agg_kernel


_R = 1000


def _row_specs(n, shapes):
    specs = []
    for shp in shapes:
        if shp[0] == n:
            specs.append(pl.BlockSpec((_R, shp[1]), lambda i: (i, 0)))
        else:
            specs.append(pl.BlockSpec(shp, lambda i: (0, 0)))
    return specs


def _tc_matmul(x, w):
    n = x.shape[0]
    d_out = w.shape[1]

    def body(x_ref, w_ref, o_ref):
        o_ref[...] = jnp.dot(x_ref[...], w_ref[...], precision=_HIGH,
                             preferred_element_type=jnp.float32)

    return pl.pallas_call(
        body,
        grid=(n // _R,),
        in_specs=_row_specs(n, [x.shape, w.shape]),
        out_specs=pl.BlockSpec((_R, d_out), lambda i: (i, 0)),
        out_shape=jax.ShapeDtypeStruct((n, d_out), jnp.float32),
    )(x, w)


def _tc_scale(h, deg0, deg1):
    n, d_out = h.shape

    def body(h_ref, d0_ref, d1_ref, dinv_ref, hs_ref):
        dinv = lax.rsqrt(d0_ref[...] + d1_ref[...] + 1.0)
        dinv_ref[...] = dinv
        hs_ref[...] = h_ref[...] * dinv

    return pl.pallas_call(
        body,
        grid=(n // _R,),
        in_specs=_row_specs(n, [h.shape, deg0.shape, deg1.shape]),
        out_specs=(
            pl.BlockSpec((_R, 1), lambda i: (i, 0)),
            pl.BlockSpec((_R, d_out), lambda i: (i, 0)),
        ),
        out_shape=(
            jax.ShapeDtypeStruct((n, 1), jnp.float32),
            jax.ShapeDtypeStruct((n, d_out), jnp.float32),
        ),
    )(h, deg0, deg1)


def _tc_mid(agg0, agg1, hs, dinv, b1, w2):
    n, d = hs.shape

    def body(a0_ref, a1_ref, hs_ref, dinv_ref, b1_ref, w2_ref, o_ref):
        z = dinv_ref[...] * (a0_ref[...] + a1_ref[...] + hs_ref[...])
        z = jnp.maximum(z + b1_ref[...], 0.0)
        o_ref[...] = dinv_ref[...] * jnp.dot(
            z, w2_ref[...], precision=_HIGH, preferred_element_type=jnp.float32
        )

    return pl.pallas_call(
        body,
        grid=(n // _R,),
        in_specs=_row_specs(
            n, [agg0.shape, agg1.shape, hs.shape, dinv.shape, b1.shape,
                w2.shape]),
        out_specs=pl.BlockSpec((_R, w2.shape[1]), lambda i: (i, 0)),
        out_shape=jax.ShapeDtypeStruct((n, w2.shape[1]), jnp.float32),
    )(agg0, agg1, hs, dinv, b1, w2)


def _tc_final(agg0, agg1, h2s, dinv, b2):
    n = h2s.shape[0]
    n_cls = b2.shape[1]

    def body(a0_ref, a1_ref, hs_ref, dinv_ref, b2_ref, o_ref):
        o = dinv_ref[...] * (
            a0_ref[:, :n_cls] + a1_ref[:, :n_cls] + hs_ref[:, :n_cls]
        )
        o = o + b2_ref[...]
        m = jnp.max(o, axis=1, keepdims=True)
        sh = o - m
        lse = jnp.log(jnp.sum(jnp.exp(sh), axis=1, keepdims=True))
        o_ref[...] = sh - lse

    return pl.pallas_call(
        body,
        grid=(n // _R,),
        in_specs=_row_specs(
            n, [agg0.shape, agg1.shape, h2s.shape, dinv.shape, b2.shape]),
        out_specs=pl.BlockSpec((_R, n_cls), lambda i: (i, 0)),
        out_shape=jax.ShapeDtypeStruct((n, n_cls), jnp.float32),
    )(agg0, agg1, h2s, dinv, b2)


def kernel(x, edge_index, W1, b1, W2, b2):
    n, d_in = x.shape
    e_total = edge_index.shape[1]
    d_hid = W1.shape[1]
    n_cls = W2.shape[1]

    chunk = 80
    src3 = edge_index[0].astype(jnp.int32).reshape(NW, -1, chunk)
    dst3 = edge_index[1].astype(jnp.int32).reshape(NW, -1, chunk)

    deg_k = _make_deg_kernel(e_total, n)
    agg1_k = _make_agg_kernel(e_total, n, d_hid)
    agg2_k = _make_agg_kernel(e_total, n, n_cls)

    h1 = _tc_matmul(x, W1)
    degp = deg_k(dst3)
    dinv, h1s = _tc_scale(h1, degp[0].reshape(n, 1), degp[1].reshape(n, 1))
    aggp1 = agg1_k(h1s, src3, dst3)
    h2s = _tc_mid(aggp1[0], aggp1[1], h1s, dinv, b1.reshape(1, d_hid), W2)
    aggp2 = agg2_k(h2s, src3, dst3)
    return _tc_final(aggp2[0], aggp2[1], h2s, dinv, b2.reshape(1, n_cls))


# speedup vs baseline: 1.4120x; 1.0075x over previous
import functools

import jax
import jax.numpy as jnp
from jax import lax
from jax.experimental import pallas as pl
from jax.experimental.pallas import tpu as pltpu
from jax.experimental.pallas import tpu_sc as plsc

NC = 2
NS = 16
L = 16
NW = NC * NS

_HIGH = jax.lax.Precision.HIGHEST


def _mesh():
    return plsc.VectorSubcoreMesh(
        core_axis_name="c", subcore_axis_name="s", num_cores=NC, num_subcores=NS
    )


def _fill1d(ref, n, value):
    v = jnp.full((L,), value, jnp.float32)

    def body(i, carry):
        ref[pl.ds(i * L, L)] = v
        return carry

    lax.fori_loop(0, n // L, body, 0)


def _fill2d(ref, rows, cols, value):
    v = jnp.full((L,), value, jnp.float32)
    per_row = cols // L

    def body(i, carry):
        r = i // per_row
        j = i % per_row
        ref[r, pl.ds(j * L, L)] = v
        return carry

    lax.fori_loop(0, rows * per_row, body, 0)


def _make_deg_kernel(e_total, n_nodes):
    ept = e_total // NW
    chunk = 80
    assert ept % chunk == 0
    nchunks = ept // chunk
    zlen = 2000
    assert n_nodes % zlen == 0
    K = 8

    @functools.partial(
        pl.kernel,
        out_type=jax.ShapeDtypeStruct((NC, n_nodes), jnp.float32),
        mesh=_mesh(),
        compiler_params=pltpu.CompilerParams(use_tc_tiling_on_sc=False),
        scratch_types=[
            pltpu.VMEM((nchunks, chunk), jnp.int32),
            pltpu.VMEM((chunk,), jnp.float32),
            pltpu.VMEM((zlen,), jnp.float32),
            pltpu.VMEM_SHARED((n_nodes,), jnp.float32),
            pltpu.SemaphoreType.DMA,
        ],
    )
    def deg_kernel(dst3_hbm, out_hbm, di_all, ones, zbuf, acc, ssem):
        c = lax.axis_index("c")
        s = lax.axis_index("s")
        wid = c * NS + s

        _fill1d(ones, chunk, 1.0)

        @pl.when(s == 0)
        def _():
            _fill1d(zbuf, zlen, 0.0)

            def zcp(k, carry):
                pltpu.sync_copy(zbuf, acc.at[pl.ds(k * zlen, zlen)])
                return carry

            lax.fori_loop(0, n_nodes // zlen, zcp, 0)

        pltpu.sync_copy(dst3_hbm.at[wid], di_all)
        plsc.subcore_barrier()

        def body(g, carry):
            pltpu.async_copy(ones, acc.at[di_all.at[g]], ssem, add=True)

            @pl.when(g >= K)
            def _():
                pltpu.make_async_copy(ones, acc.at[di_all.at[0]], ssem).wait()

            return carry

        lax.fori_loop(0, nchunks, body, 0)
        for _ in range(min(K, nchunks)):
            pltpu.make_async_copy(ones, acc.at[di_all.at[0]], ssem).wait()

        plsc.subcore_barrier()

        @pl.when(s == 0)
        def _():
            pltpu.sync_copy(acc, out_hbm.at[c])

    return deg_kernel


def _make_agg_kernel(e_total, n_nodes, d, tc_tiling=False):
    ept = e_total // NW
    chunk = 80
    assert ept % chunk == 0
    nchunks = ept // chunk
    nb = 3 if d > 64 else 5
    ngroups = nchunks // nb
    n_rchunks = n_nodes // chunk
    rounds = (n_rchunks + NS - 1) // NS

    @functools.partial(
        pl.kernel,
        out_type=jax.ShapeDtypeStruct((NC, n_nodes, d), jnp.float32),
        mesh=_mesh(),
        compiler_params=pltpu.CompilerParams(use_tc_tiling_on_sc=tc_tiling),
        scratch_types=[
            pltpu.VMEM((nchunks, chunk), jnp.int32),
            pltpu.VMEM((nchunks, chunk), jnp.int32),
            [pltpu.VMEM((chunk, d), jnp.float32) for _ in range(nb)],
            pltpu.VMEM_SHARED((n_nodes, d), jnp.float32),
            [pltpu.SemaphoreType.DMA for _ in range(nb)],
            [pltpu.SemaphoreType.DMA for _ in range(nb)],
        ],
    )
    def agg_kernel(h_hbm, src3_hbm, dst3_hbm, out_hbm, si_all, di_all, rows,
                   acc, gsem, ssem):
        c = lax.axis_index("c")
        s = lax.axis_index("s")
        wid = c * NS + s

        _fill2d(rows[0], chunk, d, 0.0)

        def zcp(k, carry):
            cid = s + k * NS

            @pl.when(cid < n_rchunks)
            def _():
                pltpu.sync_copy(rows[0], acc.at[pl.ds(cid * chunk, chunk)])

            return carry

        lax.fori_loop(0, rounds, zcp, 0)
        pltpu.sync_copy(src3_hbm.at[wid], si_all)
        pltpu.sync_copy(dst3_hbm.at[wid], di_all)
        plsc.subcore_barrier()

        for b in range(nb):
            pltpu.async_copy(h_hbm.at[si_all.at[b]], rows[b], gsem[b])

        def visit(g, b):
            pltpu.make_async_copy(
                h_hbm.at[si_all.at[0]], rows[b], gsem[b]).wait()
            pltpu.async_copy(rows[b], acc.at[di_all.at[g]], ssem[b],
                             add=True)

            @pl.when(g + nb < nchunks)
            def _():
                pltpu.make_async_copy(
                    rows[b], acc.at[di_all.at[0]], ssem[b]).wait()
                pltpu.async_copy(
                    h_hbm.at[si_all.at[g + nb]], rows[b], gsem[b])

        def body(t, carry):
            for b in range(nb):
                visit(t * nb + b, b)
            return carry

        lax.fori_loop(0, ngroups, body, 0)
        for i in range(nchunks % nb):
            visit(ngroups * nb + i, i)

        for b in range(nb):
            pltpu.make_async_copy(rows[b], acc.at[di_all.at[0]], ssem[b]).wait()

        plsc.subcore_barrier()

        def ocp(k, carry):
            cid = s + k * NS

            @pl.when(cid < n_rchunks)
            def _():
                r0 = cid * chunk
                pltpu.sync_copy(
                    acc.at[pl.ds(r0, chunk)], out_hbm.at[c, pl.ds(r0, chunk)]
                )

            return carry

        lax.fori_loop(0, rounds, ocp, 0)

    return agg_kernel


_R = 1000


def _row_specs(n, shapes):
    specs = []
    for shp in shapes:
        if shp[0] == n:
            specs.append(pl.BlockSpec((_R, shp[1]), lambda i: (i, 0)))
        else:
            specs.append(pl.BlockSpec(shp, lambda i: (0, 0)))
    return specs


def _tc_prep(x, w1, deg0, deg1):
    n = x.shape[0]
    d_out = w1.shape[1]

    def body(x_ref, w_ref, d0_ref, d1_ref, dinv_ref, hs_ref):
        dinv = lax.rsqrt(d0_ref[...] + d1_ref[...] + 1.0)
        dinv_ref[...] = dinv
        h = jnp.dot(x_ref[...], w_ref[...], precision=_HIGH,
                    preferred_element_type=jnp.float32)
        hs_ref[...] = h * dinv

    return pl.pallas_call(
        body,
        grid=(n // _R,),
        in_specs=_row_specs(n, [x.shape, w1.shape, deg0.shape, deg1.shape]),
        out_specs=(
            pl.BlockSpec((_R, 1), lambda i: (i, 0)),
            pl.BlockSpec((_R, d_out), lambda i: (i, 0)),
        ),
        out_shape=(
            jax.ShapeDtypeStruct((n, 1), jnp.float32),
            jax.ShapeDtypeStruct((n, d_out), jnp.float32),
        ),
    )(x, w1, deg0, deg1)


def _tc_mid(agg0, agg1, hs, dinv, b1, w2):
    n, d = hs.shape

    def body(a0_ref, a1_ref, hs_ref, dinv_ref, b1_ref, w2_ref, o_ref):
        z = dinv_ref[...] * (a0_ref[...] + a1_ref[...] + hs_ref[...])
        z = jnp.maximum(z + b1_ref[...], 0.0)
        o_ref[...] = dinv_ref[...] * jnp.dot(
            z, w2_ref[...], precision=_HIGH, preferred_element_type=jnp.float32
        )

    return pl.pallas_call(
        body,
        grid=(n // _R,),
        in_specs=_row_specs(
            n, [agg0.shape, agg1.shape, hs.shape, dinv.shape, b1.shape,
                w2.shape]),
        out_specs=pl.BlockSpec((_R, w2.shape[1]), lambda i: (i, 0)),
        out_shape=jax.ShapeDtypeStruct((n, w2.shape[1]), jnp.float32),
    )(agg0, agg1, hs, dinv, b1, w2)


def _tc_final(agg0, agg1, h2s, dinv, b2):
    n = h2s.shape[0]
    n_cls = b2.shape[1]

    def body(a0_ref, a1_ref, hs_ref, dinv_ref, b2_ref, o_ref):
        o = dinv_ref[...] * (
            a0_ref[:, :n_cls] + a1_ref[:, :n_cls] + hs_ref[:, :n_cls]
        )
        o = o + b2_ref[...]
        m = jnp.max(o, axis=1, keepdims=True)
        sh = o - m
        lse = jnp.log(jnp.sum(jnp.exp(sh), axis=1, keepdims=True))
        o_ref[...] = sh - lse

    return pl.pallas_call(
        body,
        grid=(n // _R,),
        in_specs=_row_specs(
            n, [agg0.shape, agg1.shape, h2s.shape, dinv.shape, b2.shape]),
        out_specs=pl.BlockSpec((_R, n_cls), lambda i: (i, 0)),
        out_shape=jax.ShapeDtypeStruct((n, n_cls), jnp.float32),
    )(agg0, agg1, h2s, dinv, b2)


def kernel(x, edge_index, W1, b1, W2, b2):
    n, d_in = x.shape
    e_total = edge_index.shape[1]
    d_hid = W1.shape[1]
    n_cls = W2.shape[1]

    chunk = 80
    src3 = edge_index[0].astype(jnp.int32).reshape(NW, -1, chunk)
    dst3 = edge_index[1].astype(jnp.int32).reshape(NW, -1, chunk)

    deg_k = _make_deg_kernel(e_total, n)
    agg1_k = _make_agg_kernel(e_total, n, d_hid)
    agg2_k = _make_agg_kernel(e_total, n, n_cls)

    degp = deg_k(dst3)
    dinv, h1s = _tc_prep(x, W1, degp[0].reshape(n, 1), degp[1].reshape(n, 1))
    aggp1 = agg1_k(h1s, src3, dst3)
    h2s = _tc_mid(aggp1[0], aggp1[1], h1s, dinv, b1.reshape(1, d_hid), W2)
    aggp2 = agg2_k(h2s, src3, dst3)
    return _tc_final(aggp2[0], aggp2[1], h2s, dinv, b2.reshape(1, n_cls))


# speedup vs baseline: 1.4609x; 1.0346x over previous
import functools

import jax
import jax.numpy as jnp
from jax import lax
from jax.experimental import pallas as pl
from jax.experimental.pallas import tpu as pltpu
from jax.experimental.pallas import tpu_sc as plsc

NC = 2
NS = 16
L = 16
NW = NC * NS

_HIGH = jax.lax.Precision.HIGHEST


def _mesh():
    return plsc.VectorSubcoreMesh(
        core_axis_name="c", subcore_axis_name="s", num_cores=NC, num_subcores=NS
    )


def _fill1d(ref, n, value):
    v = jnp.full((L,), value, jnp.float32)

    def body(i, carry):
        ref[pl.ds(i * L, L)] = v
        return carry

    lax.fori_loop(0, n // L, body, 0)


def _fill2d(ref, rows, cols, value):
    v = jnp.full((L,), value, jnp.float32)
    per_row = cols // L

    def body(i, carry):
        r = i // per_row
        j = i % per_row
        ref[r, pl.ds(j * L, L)] = v
        return carry

    lax.fori_loop(0, rows * per_row, body, 0)


def _make_deg_kernel(e_total, n_nodes):
    ept = e_total // NW
    chunk = 80
    assert ept % chunk == 0
    nchunks = ept // chunk
    zlen = 2000
    assert n_nodes % zlen == 0
    K = 8

    @functools.partial(
        pl.kernel,
        out_type=jax.ShapeDtypeStruct((NC, n_nodes), jnp.float32),
        mesh=_mesh(),
        compiler_params=pltpu.CompilerParams(use_tc_tiling_on_sc=False),
        scratch_types=[
            pltpu.VMEM((nchunks, chunk), jnp.int32),
            pltpu.VMEM((chunk,), jnp.float32),
            pltpu.VMEM((zlen,), jnp.float32),
            pltpu.VMEM_SHARED((n_nodes,), jnp.float32),
            pltpu.SemaphoreType.DMA,
        ],
    )
    def deg_kernel(dst3_hbm, out_hbm, di_all, ones, zbuf, acc, ssem):
        c = lax.axis_index("c")
        s = lax.axis_index("s")
        wid = c * NS + s

        _fill1d(ones, chunk, 1.0)

        @pl.when(s == 0)
        def _():
            _fill1d(zbuf, zlen, 0.0)

            def zcp(k, carry):
                pltpu.sync_copy(zbuf, acc.at[pl.ds(k * zlen, zlen)])
                return carry

            lax.fori_loop(0, n_nodes // zlen, zcp, 0)

        pltpu.sync_copy(dst3_hbm.at[wid], di_all)
        plsc.subcore_barrier()

        def body(g, carry):
            pltpu.async_copy(ones, acc.at[di_all.at[g]], ssem, add=True)

            @pl.when(g >= K)
            def _():
                pltpu.make_async_copy(ones, acc.at[di_all.at[0]], ssem).wait()

            return carry

        lax.fori_loop(0, nchunks, body, 0)
        for _ in range(min(K, nchunks)):
            pltpu.make_async_copy(ones, acc.at[di_all.at[0]], ssem).wait()

        plsc.subcore_barrier()

        @pl.when(s == 0)
        def _():
            pltpu.sync_copy(acc, out_hbm.at[c])

    return deg_kernel


def _make_agg_kernel(e_total, n_nodes, d, tc_tiling=False):
    ept = e_total // NW
    chunk = 80
    assert ept % chunk == 0
    nchunks = ept // chunk
    nb = 3 if d > 64 else 5
    ngroups = nchunks // nb
    n_rchunks = n_nodes // chunk
    rounds = (n_rchunks + NS - 1) // NS

    @functools.partial(
        pl.kernel,
        out_type=jax.ShapeDtypeStruct((NC, n_nodes, d), jnp.float32),
        mesh=_mesh(),
        compiler_params=pltpu.CompilerParams(use_tc_tiling_on_sc=tc_tiling),
        scratch_types=[
            pltpu.VMEM((nchunks, chunk), jnp.int32),
            pltpu.VMEM((nchunks, chunk), jnp.int32),
            [pltpu.VMEM((chunk, d), jnp.float32) for _ in range(nb)],
            pltpu.VMEM_SHARED((n_nodes, d), jnp.float32),
            [pltpu.SemaphoreType.DMA for _ in range(nb)],
            [pltpu.SemaphoreType.DMA for _ in range(nb)],
        ],
    )
    def agg_kernel(h_hbm, src3_hbm, dst3_hbm, out_hbm, si_all, di_all, rows,
                   acc, gsem, ssem):
        c = lax.axis_index("c")
        s = lax.axis_index("s")
        wid = c * NS + s

        _fill2d(rows[0], chunk, d, 0.0)

        def zcp(k, carry):
            cid = s + k * NS

            @pl.when(cid < n_rchunks)
            def _():
                pltpu.sync_copy(rows[0], acc.at[pl.ds(cid * chunk, chunk)])

            return carry

        lax.fori_loop(0, rounds, zcp, 0)
        pltpu.sync_copy(src3_hbm.at[wid], si_all)
        pltpu.sync_copy(dst3_hbm.at[wid], di_all)
        plsc.subcore_barrier()

        for b in range(nb):
            pltpu.async_copy(h_hbm.at[si_all.at[b]], rows[b], gsem[b])

        def visit(g, b):
            pltpu.make_async_copy(
                h_hbm.at[si_all.at[0]], rows[b], gsem[b]).wait()
            pltpu.async_copy(rows[b], acc.at[di_all.at[g]], ssem[b],
                             add=True)

            @pl.when(g + nb < nchunks)
            def _():
                pltpu.make_async_copy(
                    rows[b], acc.at[di_all.at[0]], ssem[b]).wait()
                pltpu.async_copy(
                    h_hbm.at[si_all.at[g + nb]], rows[b], gsem[b])

        def body(t, carry):
            for b in range(nb):
                visit(t * nb + b, b)
            return carry

        lax.fori_loop(0, ngroups, body, 0)
        for i in range(nchunks % nb):
            visit(ngroups * nb + i, i)

        for b in range(nb):
            pltpu.make_async_copy(rows[b], acc.at[di_all.at[0]], ssem[b]).wait()

        plsc.subcore_barrier()

        def ocp(k, carry):
            cid = s + k * NS

            @pl.when(cid < n_rchunks)
            def _():
                r0 = cid * chunk
                pltpu.sync_copy(
                    acc.at[pl.ds(r0, chunk)], out_hbm.at[c, pl.ds(r0, chunk)]
                )

            return carry

        lax.fori_loop(0, rounds, ocp, 0)

    return agg_kernel


_R = 2000


def _row_specs(n, shapes):
    specs = []
    for shp in shapes:
        if shp[0] == n:
            specs.append(pl.BlockSpec((_R, shp[1]), lambda i: (i, 0)))
        else:
            specs.append(pl.BlockSpec(shp, lambda i: (0, 0)))
    return specs


def _tc_prep(x, w1, deg0, deg1):
    n = x.shape[0]
    d_out = w1.shape[1]

    def body(x_ref, w_ref, d0_ref, d1_ref, dinv_ref, hs_ref):
        dinv = lax.rsqrt(d0_ref[...] + d1_ref[...] + 1.0)
        dinv_ref[...] = dinv
        h = jnp.dot(x_ref[...], w_ref[...], precision=_HIGH,
                    preferred_element_type=jnp.float32)
        hs_ref[...] = h * dinv

    return pl.pallas_call(
        body,
        grid=(n // _R,),
        in_specs=_row_specs(n, [x.shape, w1.shape, deg0.shape, deg1.shape]),
        out_specs=(
            pl.BlockSpec((_R, 1), lambda i: (i, 0)),
            pl.BlockSpec((_R, d_out), lambda i: (i, 0)),
        ),
        out_shape=(
            jax.ShapeDtypeStruct((n, 1), jnp.float32),
            jax.ShapeDtypeStruct((n, d_out), jnp.float32),
        ),
    )(x, w1, deg0, deg1)


def _tc_mid(agg0, agg1, hs, dinv, b1, w2):
    n, d = hs.shape

    def body(a0_ref, a1_ref, hs_ref, dinv_ref, b1_ref, w2_ref, o_ref):
        z = dinv_ref[...] * (a0_ref[...] + a1_ref[...] + hs_ref[...])
        z = jnp.maximum(z + b1_ref[...], 0.0)
        o_ref[...] = dinv_ref[...] * jnp.dot(
            z, w2_ref[...], precision=_HIGH, preferred_element_type=jnp.float32
        )

    return pl.pallas_call(
        body,
        grid=(n // _R,),
        in_specs=_row_specs(
            n, [agg0.shape, agg1.shape, hs.shape, dinv.shape, b1.shape,
                w2.shape]),
        out_specs=pl.BlockSpec((_R, w2.shape[1]), lambda i: (i, 0)),
        out_shape=jax.ShapeDtypeStruct((n, w2.shape[1]), jnp.float32),
    )(agg0, agg1, hs, dinv, b1, w2)


def _tc_final(agg0, agg1, h2s, dinv, b2):
    n = h2s.shape[0]
    n_cls = b2.shape[1]

    def body(a0_ref, a1_ref, hs_ref, dinv_ref, b2_ref, o_ref):
        o = dinv_ref[...] * (
            a0_ref[:, :n_cls] + a1_ref[:, :n_cls] + hs_ref[:, :n_cls]
        )
        o = o + b2_ref[...]
        m = jnp.max(o, axis=1, keepdims=True)
        sh = o - m
        lse = jnp.log(jnp.sum(jnp.exp(sh), axis=1, keepdims=True))
        o_ref[...] = sh - lse

    return pl.pallas_call(
        body,
        grid=(n // _R,),
        in_specs=_row_specs(
            n, [agg0.shape, agg1.shape, h2s.shape, dinv.shape, b2.shape]),
        out_specs=pl.BlockSpec((_R, n_cls), lambda i: (i, 0)),
        out_shape=jax.ShapeDtypeStruct((n, n_cls), jnp.float32),
    )(agg0, agg1, h2s, dinv, b2)


def kernel(x, edge_index, W1, b1, W2, b2):
    n, d_in = x.shape
    e_total = edge_index.shape[1]
    d_hid = W1.shape[1]
    n_cls = W2.shape[1]

    chunk = 80
    src3 = edge_index[0].astype(jnp.int32).reshape(NW, -1, chunk)
    dst3 = edge_index[1].astype(jnp.int32).reshape(NW, -1, chunk)

    deg_k = _make_deg_kernel(e_total, n)
    agg1_k = _make_agg_kernel(e_total, n, d_hid)
    agg2_k = _make_agg_kernel(e_total, n, n_cls)

    degp = deg_k(dst3)
    dinv, h1s = _tc_prep(x, W1, degp[0].reshape(n, 1), degp[1].reshape(n, 1))
    aggp1 = agg1_k(h1s, src3, dst3)
    h2s = _tc_mid(aggp1[0], aggp1[1], h1s, dinv, b1.reshape(1, d_hid), W2)
    aggp2 = agg2_k(h2s, src3, dst3)
    return _tc_final(aggp2[0], aggp2[1], h2s, dinv, b2.reshape(1, n_cls))
